# baseline (speedup 1.0000x reference)
import functools

import jax
import jax.numpy as jnp
from jax import lax
from jax.experimental import pallas as pl
from jax.experimental.pallas import tpu as pltpu
from jax.experimental.pallas import tpu_sc as plsc

G = 128
V = G * G * G
B, N, P = 4, 4096, 64
T = B * N * P
C = 1024
NCHUNK = T // C
NCORES = 2
NSUB = 16
NW = NCORES * NSUB
CPW = NCHUNK // NW
LANES = 16


def _softplus_body(x_ref, o_ref):
    o_ref[...] = jnp.logaddexp(x_ref[...], 0.0)


def _sigmoid_body(x_ref, o_ref):
    o_ref[...] = jax.nn.sigmoid(x_ref[...])


def _activate(x2d, body):
    rows, cols = x2d.shape
    blk = 512
    return pl.pallas_call(
        body,
        grid=(rows // blk,),
        in_specs=[pl.BlockSpec((blk, cols), lambda g: (g, 0))],
        out_specs=pl.BlockSpec((blk, cols), lambda g: (g, 0)),
        out_shape=jax.ShapeDtypeStruct((rows, cols), jnp.float32),
    )(x2d)


_ROWS_PER_STEP = 64


def _points_body(ox, oy, oz, dx, dy, dz, ln, idx_ref, w_ref):
    t = ln[...]
    x = ox[...] + dx[...] * t
    y = oy[...] + dy[...] * t
    z = oz[...] + dz[...] * t
    gf = jnp.float32(G)
    ix = ((x + 1.0) * gf - 1.0) * 0.5
    iy = ((y + 1.0) * gf - 1.0) * 0.5
    iz = ((z + 1.0) * gf - 1.0) * 0.5
    ix0 = jnp.floor(ix)
    iy0 = jnp.floor(iy)
    iz0 = jnp.floor(iz)
    fx = ix - ix0
    fy = iy - iy0
    fz = iz - iz0
    shape3 = (_ROWS_PER_STEP // 8, 8, G)
    x0 = jnp.clip(ix0, -1, G - 1).astype(jnp.int32)
    y0 = jnp.clip(iy0, -1, G - 1).astype(jnp.int32)
    for zh in (0, 1):
        zc = jnp.clip(iz0 + zh, 0, G - 1).astype(jnp.int32)
        j = (zc * G + y0) * G + x0 + (G + 1)
        idx_ref[:, zh] = j.reshape(shape3)
    for a in (0, 1):
        for b in (0, 1):
            for c in (0, 1):
                k = a * 4 + b * 2 + c
                zi = iz0 + a
                yi = iy0 + b
                xi = ix0 + c
                wz = fz if a else 1.0 - fz
                wy = fy if b else 1.0 - fy
                wx = fx if c else 1.0 - fx
                valid = ((xi >= 0) & (xi < gf) & (yi >= 0) & (yi < gf)
                         & (zi >= 0) & (zi < gf))
                w_ref[:, k] = jnp.where(valid, wx * wy * wz, 0.0).reshape(shape3)


def _point_math(ox, oy, oz, dx, dy, dz, ln):
    nrows = T // G
    nsteps = nrows // _ROWS_PER_STEP
    cps = _ROWS_PER_STEP // 8
    in_spec = pl.BlockSpec((_ROWS_PER_STEP, G), lambda g: (g, 0))
    return pl.pallas_call(
        _points_body,
        grid=(nsteps,),
        in_specs=[in_spec] * 7,
        out_specs=[
            pl.BlockSpec((cps, 2, 8, G), lambda g: (g, 0, 0, 0)),
            pl.BlockSpec((cps, 8, 8, G), lambda g: (g, 0, 0, 0)),
        ],
        out_shape=[
            jax.ShapeDtypeStruct((NCHUNK, 2, 8, G), jnp.int32),
            jax.ShapeDtypeStruct((NCHUNK, 8, 8, G), jnp.float32),
        ],
    )(ox, oy, oz, dx, dy, dz, ln)


RB = 2048
MP = NW * 33 * RB
RPW = MP // NW
CH = 2184


def _sc_table_body(p0_hbm, p1_hbm, p2_hbm, p3_hbm, tbl_hbm, stage_v, tbl_v,
                   sem):
    wid = lax.axis_index("s") * NCORES + lax.axis_index("c")
    lanes = lax.iota(jnp.int32, LANES)
    ch_l = lanes & 3
    off_l = (lanes >> 3) * G + ((lanes >> 2) & 1)
    cvec = ch_l * CH + off_l
    srcs = (p0_hbm, p1_hbm, p2_hbm, p3_hbm)
    r0w = wid * RPW

    @pl.loop(0, RPW + RB, step=2 * RB)
    def _blk2(rr0):
        for b in (0, 1):
            rr = rr0 + b * RB

            @pl.when(rr < RPW)
            def _():
                r0 = r0w + rr

                @pl.when(rr0 > 0)
                def _():
                    pltpu.make_async_copy(
                        tbl_v.at[b], tbl_hbm.at[pl.ds(0, RB * LANES)],
                        sem).wait()

                for c4 in range(4):
                    pltpu.sync_copy(srcs[c4].at[pl.ds(r0, CH)],
                                    stage_v.at[pl.ds(c4 * CH, CH)])

                @plsc.parallel_loop(0, RB, step=1, unroll=4)
                def _row(j2):
                    row = plsc.load_gather(stage_v, [cvec + j2])
                    tbl_v[b, pl.ds(j2 * LANES, LANES)] = row

                pltpu.async_copy(tbl_v.at[b],
                                 tbl_hbm.at[pl.ds(r0 * LANES, RB * LANES)],
                                 sem)

    for _ in range(2):
        pltpu.make_async_copy(
            tbl_v.at[0], tbl_hbm.at[pl.ds(0, RB * LANES)], sem).wait()


@functools.lru_cache(maxsize=None)
def _sc_table():
    return pl.kernel(
        _sc_table_body,
        out_type=jax.ShapeDtypeStruct((MP * LANES,), jnp.float32),
        mesh=plsc.VectorSubcoreMesh(
            core_axis_name="c", subcore_axis_name="s",
            num_cores=NCORES, num_subcores=NSUB),
        compiler_params=pltpu.CompilerParams(
            needs_layout_passes=False, use_tc_tiling_on_sc=False),
        scratch_types=[
            pltpu.VMEM((4 * CH,), jnp.float32),
            pltpu.VMEM((2, RB * LANES), jnp.float32),
            pltpu.SemaphoreType.DMA,
        ],
    )


def _lane_perm(x, idx):
    return lax.gather(
        x, idx[:, None],
        lax.GatherDimensionNumbers(
            offset_dims=(), collapsed_slice_dims=(0,), start_index_map=(0,)),
        slice_sizes=(1,),
        mode=lax.GatherScatterMode.PROMISE_IN_BOUNDS)


def _sc_body(table_hbm, idx_hbm, w_hbm,
             out_d, out_r, out_g, out_b, idx_v, ws_v, w_v, g_v, o4_v, sem):
    wid = lax.axis_index("s") * NCORES + lax.axis_index("c")
    lanes = lax.iota(jnp.int32, LANES)
    perm4 = (lanes + 4) & 15
    perm8 = (lanes + 8) & 15
    permw = lanes >> 2
    pats = [((lanes & 3) << 2) + ch for ch in range(4)]
    l9 = lanes * 9
    m4 = lanes < 4
    m8 = lanes < 8
    m12 = lanes < 12
    outs = (out_d, out_r, out_g, out_b)

    def _fetch(jc, b):
        g = wid * CPW + jc
        pltpu.sync_copy(idx_hbm.at[pl.ds(g * 2 * C, 2 * C)], idx_v.at[b])
        pltpu.async_copy(table_hbm.at[idx_v.at[b]], g_v.at[b], sem)

    _fetch(0, 0)

    @pl.loop(0, CPW, step=2)
    def _chunk2(jc0):
        for b in (0, 1):
            jc = jc0 + b
            g = wid * CPW + jc
            g_b = g_v.at[b]
            pltpu.make_async_copy(table_hbm.at[idx_v.at[b]], g_b, sem).wait()

            @pl.when(jc + 1 < CPW)
            def _():
                _fetch(jc + 1, 1 - b)

            pltpu.sync_copy(w_hbm.at[pl.ds(g * 8 * C, 8 * C)], ws_v)
            _compute_chunk(g, g_b, ws_v, w_v, o4_v, outs,
                           perm4, perm8, permw, pats, l9, m4, m8, m12)


def _compute_chunk(g, g_b, ws_v, w_v, o4_v, outs,
                   perm4, perm8, permw, pats, l9, m4, m8, m12):
    if True:
        @plsc.parallel_loop(0, C, step=LANES)
        def _tr(p0):
            for k in range(8):
                v = ws_v[pl.ds(k * C + p0, LANES)]
                plsc.store_scatter(w_v, [l9 + (9 * p0 + k)], v)

        @plsc.parallel_loop(0, C, step=16)
        def _pts(p):
            combs = []
            for q4 in range(4):
                t2 = []
                for q in range(4):
                    pq = p + q4 * 4 + q
                    a = g_b[pq, :]
                    b = g_b[C + pq, :]
                    wv = w_v[pl.ds(9 * pq, LANES)]
                    wa = _lane_perm(wv, permw)
                    wb = _lane_perm(wv, permw + 4)
                    s = a * wa + b * wb
                    t1 = s + _lane_perm(s, perm8)
                    t2.append(t1 + _lane_perm(t1, perm4))
                combs.append(jnp.where(m4, t2[0],
                                       jnp.where(m8, t2[1],
                                                 jnp.where(m12, t2[2], t2[3]))))
            for ch in range(4):
                e = [_lane_perm(cb, pats[ch]) for cb in combs]
                v16 = jnp.where(m4, e[0],
                                jnp.where(m8, e[1],
                                          jnp.where(m12, e[2], e[3])))
                o4_v[ch, pl.ds(p, LANES)] = v16

        for ch in range(4):
            pltpu.sync_copy(o4_v.at[ch], outs[ch].at[pl.ds(g * C, C)])


@functools.lru_cache(maxsize=None)
def _sc_gather():
    return pl.kernel(
        _sc_body,
        out_type=[jax.ShapeDtypeStruct((T,), jnp.float32)] * 4,
        mesh=plsc.VectorSubcoreMesh(
            core_axis_name="c", subcore_axis_name="s",
            num_cores=NCORES, num_subcores=NSUB),
        compiler_params=pltpu.CompilerParams(
            needs_layout_passes=False, use_tc_tiling_on_sc=False),
        scratch_types=[
            pltpu.VMEM((2, 2 * C), jnp.int32),
            pltpu.VMEM((8 * C,), jnp.float32),
            pltpu.VMEM((9 * C + LANES,), jnp.float32),
            pltpu.VMEM((2, 2 * C, LANES), jnp.float32),
            pltpu.VMEM((4, C), jnp.float32),
            pltpu.SemaphoreType.DMA,
        ],
    )


def kernel(density, color, origins, directions, lengths):
    dens2 = density.reshape(V // 512, 512)
    col2 = color.reshape(3 * V // 512, 512)
    d_act = _activate(dens2, _softplus_body).reshape(V)
    c_act = _activate(col2, _sigmoid_body).reshape(3, V)

    chans = (d_act, c_act[0], c_act[1], c_act[2])
    tail = MP + CH - (G + 1) - V
    pch = [jnp.concatenate([jnp.zeros((G + 1,), jnp.float32), chan,
                            jnp.zeros((tail,), jnp.float32)]) for chan in chans]
    table = _sc_table()(*pch).reshape(MP, LANES)

    ob = jnp.broadcast_to(origins[:, :, None, :], (B, N, P, 3))
    db = jnp.broadcast_to(directions[:, :, None, :], (B, N, P, 3))
    ox, oy, oz = (ob[..., i].reshape(T // G, G) for i in range(3))
    dx, dy, dz = (db[..., i].reshape(T // G, G) for i in range(3))
    ln = lengths.reshape(T // G, G)

    idx_all, w_all = _point_math(ox, oy, oz, dx, dy, dz, ln)
    idx1 = idx_all.reshape(NCHUNK * 2 * C)
    w1 = w_all.reshape(NCHUNK * 8 * C)

    d, r, gg, b = _sc_gather()(table, idx1, w1)

    d_s = d.reshape(B, N, P, 1)
    f_s = jnp.stack([r, gg, b], axis=-1).reshape(B, N, P, 3)
    return (d_s, f_s)


# speedup vs baseline: 1.6338x; 1.6338x over previous
import functools

import jax
import jax.numpy as jnp
from jax import lax
from jax.experimental import pallas as pl
from jax.experimental.pallas import tpu as pltpu
from jax.experimental.pallas import tpu_sc as plsc

G = 128
V = G * G * G
B, N, P = 4, 4096, 64
T = B * N * P
C = 1024
NCHUNK = T // C
NCORES = 2
NSUB = 16
NW = NCORES * NSUB
CPW = NCHUNK // NW
LANES = 16


def _softplus_body(x_ref, o_ref):
    o_ref[...] = jnp.logaddexp(x_ref[...], 0.0)


def _sigmoid_body(x_ref, o_ref):
    o_ref[...] = jax.nn.sigmoid(x_ref[...])


def _activate(x2d, body):
    rows, cols = x2d.shape
    blk = 512
    return pl.pallas_call(
        body,
        grid=(rows // blk,),
        in_specs=[pl.BlockSpec((blk, cols), lambda g: (g, 0))],
        out_specs=pl.BlockSpec((blk, cols), lambda g: (g, 0)),
        out_shape=jax.ShapeDtypeStruct((rows, cols), jnp.float32),
    )(x2d)


_ROWS_PER_STEP = 64


def _points_body(ox, oy, oz, dx, dy, dz, ln, idx_ref, w_ref):
    t = ln[...]
    x = ox[...] + dx[...] * t
    y = oy[...] + dy[...] * t
    z = oz[...] + dz[...] * t
    gf = jnp.float32(G)
    ix = ((x + 1.0) * gf - 1.0) * 0.5
    iy = ((y + 1.0) * gf - 1.0) * 0.5
    iz = ((z + 1.0) * gf - 1.0) * 0.5
    ix0 = jnp.floor(ix)
    iy0 = jnp.floor(iy)
    iz0 = jnp.floor(iz)
    fx = ix - ix0
    fy = iy - iy0
    fz = iz - iz0
    shape3 = (_ROWS_PER_STEP // 8, 8, G)
    x0 = jnp.clip(ix0, -1, G - 1).astype(jnp.int32)
    y0 = jnp.clip(iy0, -1, G - 1).astype(jnp.int32)
    for zh in (0, 1):
        zc = jnp.clip(iz0 + zh, 0, G - 1).astype(jnp.int32)
        j = (zc * G + y0) * G + x0 + BIAS
        idx_ref[:, zh] = j.reshape(shape3)
    for a in (0, 1):
        for b in (0, 1):
            for c in (0, 1):
                k = a * 4 + b * 2 + c
                zi = iz0 + a
                yi = iy0 + b
                xi = ix0 + c
                wz = fz if a else 1.0 - fz
                wy = fy if b else 1.0 - fy
                wx = fx if c else 1.0 - fx
                valid = ((xi >= 0) & (xi < gf) & (yi >= 0) & (yi < gf)
                         & (zi >= 0) & (zi < gf))
                w_ref[:, k] = jnp.where(valid, wx * wy * wz, 0.0).reshape(shape3)


def _point_math(ox, oy, oz, dx, dy, dz, ln):
    nrows = T // G
    nsteps = nrows // _ROWS_PER_STEP
    cps = _ROWS_PER_STEP // 8
    in_spec = pl.BlockSpec((_ROWS_PER_STEP, G), lambda g: (g, 0))
    return pl.pallas_call(
        _points_body,
        grid=(nsteps,),
        in_specs=[in_spec] * 7,
        out_specs=[
            pl.BlockSpec((cps, 2, 8, G), lambda g: (g, 0, 0, 0)),
            pl.BlockSpec((cps, 8, 8, G), lambda g: (g, 0, 0, 0)),
        ],
        out_shape=[
            jax.ShapeDtypeStruct((NCHUNK, 2, 8, G), jnp.int32),
            jax.ShapeDtypeStruct((NCHUNK, 8, 8, G), jnp.float32),
        ],
    )(ox, oy, oz, dx, dy, dz, ln)


RB = 2048
MP = NW * 33 * RB
RPW = MP // NW
CH = 2184
BIAS = 136


def _sc_table_body(p0_hbm, p1_hbm, p2_hbm, p3_hbm, tbl_hbm, stage_v, tbl_v,
                   sem):
    wid = lax.axis_index("s") * NCORES + lax.axis_index("c")
    lanes = lax.iota(jnp.int32, LANES)
    ch_l = lanes & 3
    off_l = (lanes >> 3) * G + ((lanes >> 2) & 1)
    cvec = ch_l * CH + off_l
    srcs = (p0_hbm, p1_hbm, p2_hbm, p3_hbm)
    r0w = wid * RPW

    @pl.loop(0, RPW + RB, step=2 * RB)
    def _blk2(rr0):
        for b in (0, 1):
            rr = rr0 + b * RB

            @pl.when(rr < RPW)
            def _():
                r0 = r0w + rr
                lo = r0 - BIAS

                @pl.when(rr0 > 0)
                def _():
                    pltpu.make_async_copy(
                        tbl_v.at[b], tbl_hbm.at[pl.ds(0, RB * LANES)],
                        sem).wait()

                @pl.when(r0 == 0)
                def _():
                    @plsc.parallel_loop(0, BIAS, step=LANES)
                    def _z(i):
                        for c4 in range(4):
                            stage_v[pl.ds(c4 * CH + i, LANES)] = (
                                jnp.zeros((LANES,), jnp.float32))
                    for c4 in range(4):
                        pltpu.sync_copy(
                            srcs[c4].at[pl.ds(0, CH - BIAS)],
                            stage_v.at[pl.ds(c4 * CH + BIAS, CH - BIAS)])

                @pl.when((r0 > 0) & (r0 + RB <= V))
                def _():
                    for c4 in range(4):
                        pltpu.sync_copy(srcs[c4].at[pl.ds(lo, CH)],
                                        stage_v.at[pl.ds(c4 * CH, CH)])

                @pl.when(r0 == V)
                def _():
                    @plsc.parallel_loop(0, CH, step=LANES)
                    def _z(i):
                        for c4 in range(4):
                            stage_v[pl.ds(c4 * CH + i, LANES)] = (
                                jnp.zeros((LANES,), jnp.float32))
                    for c4 in range(4):
                        pltpu.sync_copy(srcs[c4].at[pl.ds(V - BIAS, BIAS)],
                                        stage_v.at[pl.ds(c4 * CH, BIAS)])

                @plsc.parallel_loop(0, RB, step=1, unroll=4)
                def _row(j2):
                    row = plsc.load_gather(stage_v, [cvec + j2])
                    tbl_v[b, pl.ds(j2 * LANES, LANES)] = row

                pltpu.async_copy(tbl_v.at[b],
                                 tbl_hbm.at[pl.ds(r0 * LANES, RB * LANES)],
                                 sem)

    for _ in range(2):
        pltpu.make_async_copy(
            tbl_v.at[0], tbl_hbm.at[pl.ds(0, RB * LANES)], sem).wait()


@functools.lru_cache(maxsize=None)
def _sc_table():
    return pl.kernel(
        _sc_table_body,
        out_type=jax.ShapeDtypeStruct((MP * LANES,), jnp.float32),
        mesh=plsc.VectorSubcoreMesh(
            core_axis_name="c", subcore_axis_name="s",
            num_cores=NCORES, num_subcores=NSUB),
        compiler_params=pltpu.CompilerParams(
            needs_layout_passes=False, use_tc_tiling_on_sc=False),
        scratch_types=[
            pltpu.VMEM((4 * CH + LANES,), jnp.float32),
            pltpu.VMEM((2, RB * LANES), jnp.float32),
            pltpu.SemaphoreType.DMA,
        ],
    )


def _lane_perm(x, idx):
    return lax.gather(
        x, idx[:, None],
        lax.GatherDimensionNumbers(
            offset_dims=(), collapsed_slice_dims=(0,), start_index_map=(0,)),
        slice_sizes=(1,),
        mode=lax.GatherScatterMode.PROMISE_IN_BOUNDS)


def _sc_body(table_hbm, idx_hbm, w_hbm,
             out_d, out_r, out_g, out_b, idx_v, ws_v, w_v, g_v, o4_v, sem):
    wid = lax.axis_index("s") * NCORES + lax.axis_index("c")
    lanes = lax.iota(jnp.int32, LANES)
    perm4 = (lanes + 4) & 15
    perm8 = (lanes + 8) & 15
    permw = lanes >> 2
    pats = [((lanes & 3) << 2) + ch for ch in range(4)]
    l9 = lanes * 9
    m4 = lanes < 4
    m8 = lanes < 8
    m12 = lanes < 12
    outs = (out_d, out_r, out_g, out_b)

    def _fetch(jc, b):
        g = wid * CPW + jc
        pltpu.sync_copy(idx_hbm.at[pl.ds(g * 2 * C, 2 * C)], idx_v.at[b])
        pltpu.async_copy(table_hbm.at[idx_v.at[b]], g_v.at[b], sem)

    _fetch(0, 0)

    @pl.loop(0, CPW, step=2)
    def _chunk2(jc0):
        for b in (0, 1):
            jc = jc0 + b
            g = wid * CPW + jc
            g_b = g_v.at[b]
            pltpu.make_async_copy(table_hbm.at[idx_v.at[b]], g_b, sem).wait()

            @pl.when(jc + 1 < CPW)
            def _():
                _fetch(jc + 1, 1 - b)

            pltpu.sync_copy(w_hbm.at[pl.ds(g * 8 * C, 8 * C)], ws_v)
            _compute_chunk(g, g_b, ws_v, w_v, o4_v, outs,
                           perm4, perm8, permw, pats, l9, m4, m8, m12)


def _compute_chunk(g, g_b, ws_v, w_v, o4_v, outs,
                   perm4, perm8, permw, pats, l9, m4, m8, m12):
    if True:
        @plsc.parallel_loop(0, C, step=LANES)
        def _tr(p0):
            for k in range(8):
                v = ws_v[pl.ds(k * C + p0, LANES)]
                plsc.store_scatter(w_v, [l9 + (9 * p0 + k)], v)

        @plsc.parallel_loop(0, C, step=16)
        def _pts(p):
            combs = []
            for q4 in range(4):
                t2 = []
                for q in range(4):
                    pq = p + q4 * 4 + q
                    a = g_b[pq, :]
                    b = g_b[C + pq, :]
                    wv = w_v[pl.ds(9 * pq, LANES)]
                    wa = _lane_perm(wv, permw)
                    wb = _lane_perm(wv, permw + 4)
                    s = a * wa + b * wb
                    t1 = s + _lane_perm(s, perm8)
                    t2.append(t1 + _lane_perm(t1, perm4))
                combs.append(jnp.where(m4, t2[0],
                                       jnp.where(m8, t2[1],
                                                 jnp.where(m12, t2[2], t2[3]))))
            for ch in range(4):
                e = [_lane_perm(cb, pats[ch]) for cb in combs]
                v16 = jnp.where(m4, e[0],
                                jnp.where(m8, e[1],
                                          jnp.where(m12, e[2], e[3])))
                o4_v[ch, pl.ds(p, LANES)] = v16

        for ch in range(4):
            pltpu.sync_copy(o4_v.at[ch], outs[ch].at[pl.ds(g * C, C)])


@functools.lru_cache(maxsize=None)
def _sc_gather():
    return pl.kernel(
        _sc_body,
        out_type=[jax.ShapeDtypeStruct((T,), jnp.float32)] * 4,
        mesh=plsc.VectorSubcoreMesh(
            core_axis_name="c", subcore_axis_name="s",
            num_cores=NCORES, num_subcores=NSUB),
        compiler_params=pltpu.CompilerParams(
            needs_layout_passes=False, use_tc_tiling_on_sc=False),
        scratch_types=[
            pltpu.VMEM((2, 2 * C), jnp.int32),
            pltpu.VMEM((8 * C,), jnp.float32),
            pltpu.VMEM((9 * C + LANES,), jnp.float32),
            pltpu.VMEM((2, 2 * C, LANES), jnp.float32),
            pltpu.VMEM((4, C), jnp.float32),
            pltpu.SemaphoreType.DMA,
        ],
    )


def kernel(density, color, origins, directions, lengths):
    d_act = _activate(density.reshape(V // G, G), _softplus_body)
    c_act = _activate(color.reshape(3 * V // G, G),
                      _sigmoid_body).reshape(3, V // G, G)

    chans = (d_act.reshape(V), c_act[0].reshape(V),
             c_act[1].reshape(V), c_act[2].reshape(V))
    table = _sc_table()(*chans).reshape(MP, LANES)

    ob = jnp.broadcast_to(origins[:, :, None, :], (B, N, P, 3))
    db = jnp.broadcast_to(directions[:, :, None, :], (B, N, P, 3))
    ox, oy, oz = (ob[..., i].reshape(T // G, G) for i in range(3))
    dx, dy, dz = (db[..., i].reshape(T // G, G) for i in range(3))
    ln = lengths.reshape(T // G, G)

    idx_all, w_all = _point_math(ox, oy, oz, dx, dy, dz, ln)
    idx1 = idx_all.reshape(NCHUNK * 2 * C)
    w1 = w_all.reshape(NCHUNK * 8 * C)

    d, r, gg, b = _sc_gather()(table, idx1, w1)

    d_s = d.reshape(B, N, P, 1)
    f_s = jnp.stack([r, gg, b], axis=-1).reshape(B, N, P, 3)
    return (d_s, f_s)


# speedup vs baseline: 1.6359x; 1.0013x over previous
import functools

import jax
import jax.numpy as jnp
from jax import lax
from jax.experimental import pallas as pl
from jax.experimental.pallas import tpu as pltpu
from jax.experimental.pallas import tpu_sc as plsc

G = 128
V = G * G * G
B, N, P = 4, 4096, 64
T = B * N * P
C = 1024
NCHUNK = T // C
NCORES = 2
NSUB = 16
NW = NCORES * NSUB
CPW = NCHUNK // NW
LANES = 16


def _softplus_body(x_ref, o_ref):
    o_ref[...] = jnp.logaddexp(x_ref[...], 0.0)


def _sigmoid_body(x_ref, o_ref):
    o_ref[...] = jax.nn.sigmoid(x_ref[...])


def _activate(x2d, body):
    rows, cols = x2d.shape
    blk = 512
    return pl.pallas_call(
        body,
        grid=(rows // blk,),
        in_specs=[pl.BlockSpec((blk, cols), lambda g: (g, 0))],
        out_specs=pl.BlockSpec((blk, cols), lambda g: (g, 0)),
        out_shape=jax.ShapeDtypeStruct((rows, cols), jnp.float32),
    )(x2d)


_ROWS_PER_STEP = 64


def _points_body(ox, oy, oz, dx, dy, dz, ln, idx_ref, w_ref):
    t = ln[...]
    x = ox[...] + dx[...] * t
    y = oy[...] + dy[...] * t
    z = oz[...] + dz[...] * t
    gf = jnp.float32(G)
    ix = ((x + 1.0) * gf - 1.0) * 0.5
    iy = ((y + 1.0) * gf - 1.0) * 0.5
    iz = ((z + 1.0) * gf - 1.0) * 0.5
    ix0 = jnp.floor(ix)
    iy0 = jnp.floor(iy)
    iz0 = jnp.floor(iz)
    fx = ix - ix0
    fy = iy - iy0
    fz = iz - iz0
    shape3 = (_ROWS_PER_STEP // 8, 8, G)
    x0 = jnp.clip(ix0, -1, G - 1).astype(jnp.int32)
    y0 = jnp.clip(iy0, -1, G - 1).astype(jnp.int32)
    for zh in (0, 1):
        zc = jnp.clip(iz0 + zh, 0, G - 1).astype(jnp.int32)
        j = (zc * G + y0) * G + x0 + BIAS
        idx_ref[:, zh] = j.reshape(shape3)
    for a in (0, 1):
        for b in (0, 1):
            for c in (0, 1):
                k = a * 4 + b * 2 + c
                zi = iz0 + a
                yi = iy0 + b
                xi = ix0 + c
                wz = fz if a else 1.0 - fz
                wy = fy if b else 1.0 - fy
                wx = fx if c else 1.0 - fx
                valid = ((xi >= 0) & (xi < gf) & (yi >= 0) & (yi < gf)
                         & (zi >= 0) & (zi < gf))
                w_ref[:, k] = jnp.where(valid, wx * wy * wz, 0.0).reshape(shape3)


def _point_math(ox, oy, oz, dx, dy, dz, ln):
    nrows = T // G
    nsteps = nrows // _ROWS_PER_STEP
    cps = _ROWS_PER_STEP // 8
    in_spec = pl.BlockSpec((_ROWS_PER_STEP, G), lambda g: (g, 0))
    return pl.pallas_call(
        _points_body,
        grid=(nsteps,),
        in_specs=[in_spec] * 7,
        out_specs=[
            pl.BlockSpec((cps, 2, 8, G), lambda g: (g, 0, 0, 0)),
            pl.BlockSpec((cps, 8, 8, G), lambda g: (g, 0, 0, 0)),
        ],
        out_shape=[
            jax.ShapeDtypeStruct((NCHUNK, 2, 8, G), jnp.int32),
            jax.ShapeDtypeStruct((NCHUNK, 8, 8, G), jnp.float32),
        ],
    )(ox, oy, oz, dx, dy, dz, ln)


RB = 2048
MP = NW * 33 * RB
RPW = MP // NW
CH = 2184
BIAS = 136


def _sc_table_body(p0_hbm, p1_hbm, p2_hbm, p3_hbm, tbl_hbm, stage_v, tbl_v,
                   sem):
    wid = lax.axis_index("s") * NCORES + lax.axis_index("c")
    lanes = lax.iota(jnp.int32, LANES)
    ch_l = lanes & 3
    off_l = (lanes >> 3) * G + ((lanes >> 2) & 1)
    cvec = ch_l * CH + off_l
    srcs = (p0_hbm, p1_hbm, p2_hbm, p3_hbm)
    r0w = wid * RPW

    @pl.loop(0, RPW + RB, step=2 * RB)
    def _blk2(rr0):
        for b in (0, 1):
            rr = rr0 + b * RB

            @pl.when(rr < RPW)
            def _():
                r0 = r0w + rr
                lo = r0 - BIAS

                @pl.when(rr0 > 0)
                def _():
                    pltpu.make_async_copy(
                        tbl_v.at[b], tbl_hbm.at[pl.ds(0, RB * LANES)],
                        sem).wait()

                @pl.when(r0 == 0)
                def _():
                    @plsc.parallel_loop(0, BIAS, step=LANES)
                    def _z(i):
                        for c4 in range(4):
                            stage_v[pl.ds(c4 * CH + i, LANES)] = (
                                jnp.zeros((LANES,), jnp.float32))
                    for c4 in range(4):
                        pltpu.sync_copy(
                            srcs[c4].at[pl.ds(0, CH - BIAS)],
                            stage_v.at[pl.ds(c4 * CH + BIAS, CH - BIAS)])

                @pl.when((r0 > 0) & (r0 + RB <= V))
                def _():
                    for c4 in range(4):
                        pltpu.sync_copy(srcs[c4].at[pl.ds(lo, CH)],
                                        stage_v.at[pl.ds(c4 * CH, CH)])

                @pl.when(r0 == V)
                def _():
                    @plsc.parallel_loop(0, CH, step=LANES)
                    def _z(i):
                        for c4 in range(4):
                            stage_v[pl.ds(c4 * CH + i, LANES)] = (
                                jnp.zeros((LANES,), jnp.float32))
                    for c4 in range(4):
                        pltpu.sync_copy(srcs[c4].at[pl.ds(V - BIAS, BIAS)],
                                        stage_v.at[pl.ds(c4 * CH, BIAS)])

                @plsc.parallel_loop(0, RB, step=1, unroll=8)
                def _row(j2):
                    row = plsc.load_gather(stage_v, [cvec + j2])
                    tbl_v[b, pl.ds(j2 * LANES, LANES)] = row

                pltpu.async_copy(tbl_v.at[b],
                                 tbl_hbm.at[pl.ds(r0 * LANES, RB * LANES)],
                                 sem)

    for _ in range(2):
        pltpu.make_async_copy(
            tbl_v.at[0], tbl_hbm.at[pl.ds(0, RB * LANES)], sem).wait()


@functools.lru_cache(maxsize=None)
def _sc_table():
    return pl.kernel(
        _sc_table_body,
        out_type=jax.ShapeDtypeStruct((MP * LANES,), jnp.float32),
        mesh=plsc.VectorSubcoreMesh(
            core_axis_name="c", subcore_axis_name="s",
            num_cores=NCORES, num_subcores=NSUB),
        compiler_params=pltpu.CompilerParams(
            needs_layout_passes=False, use_tc_tiling_on_sc=False),
        scratch_types=[
            pltpu.VMEM((4 * CH + LANES,), jnp.float32),
            pltpu.VMEM((2, RB * LANES), jnp.float32),
            pltpu.SemaphoreType.DMA,
        ],
    )


def _lane_perm(x, idx):
    return lax.gather(
        x, idx[:, None],
        lax.GatherDimensionNumbers(
            offset_dims=(), collapsed_slice_dims=(0,), start_index_map=(0,)),
        slice_sizes=(1,),
        mode=lax.GatherScatterMode.PROMISE_IN_BOUNDS)


def _sc_body(table_hbm, idx_hbm, w_hbm,
             out_d, out_r, out_g, out_b, idx_v, ws_v, w_v, g_v, o4_v, sem):
    wid = lax.axis_index("s") * NCORES + lax.axis_index("c")
    lanes = lax.iota(jnp.int32, LANES)
    perm4 = (lanes + 4) & 15
    perm8 = (lanes + 8) & 15
    permw = lanes >> 2
    pats = [((lanes & 3) << 2) + ch for ch in range(4)]
    l9 = lanes * 9
    m4 = lanes < 4
    m8 = lanes < 8
    m12 = lanes < 12
    outs = (out_d, out_r, out_g, out_b)

    def _fetch(jc, b):
        g = wid * CPW + jc
        pltpu.sync_copy(idx_hbm.at[pl.ds(g * 2 * C, 2 * C)], idx_v.at[b])
        pltpu.async_copy(table_hbm.at[idx_v.at[b]], g_v.at[b], sem)

    _fetch(0, 0)

    @pl.loop(0, CPW, step=2)
    def _chunk2(jc0):
        for b in (0, 1):
            jc = jc0 + b
            g = wid * CPW + jc
            g_b = g_v.at[b]
            pltpu.make_async_copy(table_hbm.at[idx_v.at[b]], g_b, sem).wait()

            @pl.when(jc + 1 < CPW)
            def _():
                _fetch(jc + 1, 1 - b)

            pltpu.sync_copy(w_hbm.at[pl.ds(g * 8 * C, 8 * C)], ws_v)
            _compute_chunk(g, g_b, ws_v, w_v, o4_v, outs,
                           perm4, perm8, permw, pats, l9, m4, m8, m12)


def _compute_chunk(g, g_b, ws_v, w_v, o4_v, outs,
                   perm4, perm8, permw, pats, l9, m4, m8, m12):
    if True:
        @plsc.parallel_loop(0, C, step=LANES)
        def _tr(p0):
            for k in range(8):
                v = ws_v[pl.ds(k * C + p0, LANES)]
                plsc.store_scatter(w_v, [l9 + (9 * p0 + k)], v)

        @plsc.parallel_loop(0, C, step=16)
        def _pts(p):
            combs = []
            for q4 in range(4):
                t2 = []
                for q in range(4):
                    pq = p + q4 * 4 + q
                    a = g_b[pq, :]
                    b = g_b[C + pq, :]
                    wv = w_v[pl.ds(9 * pq, LANES)]
                    wa = _lane_perm(wv, permw)
                    wb = _lane_perm(wv, permw + 4)
                    s = a * wa + b * wb
                    t1 = s + _lane_perm(s, perm8)
                    t2.append(t1 + _lane_perm(t1, perm4))
                combs.append(jnp.where(m4, t2[0],
                                       jnp.where(m8, t2[1],
                                                 jnp.where(m12, t2[2], t2[3]))))
            for ch in range(4):
                e = [_lane_perm(cb, pats[ch]) for cb in combs]
                v16 = jnp.where(m4, e[0],
                                jnp.where(m8, e[1],
                                          jnp.where(m12, e[2], e[3])))
                o4_v[ch, pl.ds(p, LANES)] = v16

        for ch in range(4):
            pltpu.sync_copy(o4_v.at[ch], outs[ch].at[pl.ds(g * C, C)])


@functools.lru_cache(maxsize=None)
def _sc_gather():
    return pl.kernel(
        _sc_body,
        out_type=[jax.ShapeDtypeStruct((T,), jnp.float32)] * 4,
        mesh=plsc.VectorSubcoreMesh(
            core_axis_name="c", subcore_axis_name="s",
            num_cores=NCORES, num_subcores=NSUB),
        compiler_params=pltpu.CompilerParams(
            needs_layout_passes=False, use_tc_tiling_on_sc=False),
        scratch_types=[
            pltpu.VMEM((2, 2 * C), jnp.int32),
            pltpu.VMEM((8 * C,), jnp.float32),
            pltpu.VMEM((9 * C + LANES,), jnp.float32),
            pltpu.VMEM((2, 2 * C, LANES), jnp.float32),
            pltpu.VMEM((4, C), jnp.float32),
            pltpu.SemaphoreType.DMA,
        ],
    )


def kernel(density, color, origins, directions, lengths):
    d_act = _activate(density.reshape(V // G, G), _softplus_body)
    c_act = _activate(color.reshape(3 * V // G, G),
                      _sigmoid_body).reshape(3, V // G, G)

    chans = (d_act.reshape(V), c_act[0].reshape(V),
             c_act[1].reshape(V), c_act[2].reshape(V))
    table = _sc_table()(*chans).reshape(MP, LANES)

    ob = jnp.broadcast_to(origins[:, :, None, :], (B, N, P, 3))
    db = jnp.broadcast_to(directions[:, :, None, :], (B, N, P, 3))
    ox, oy, oz = (ob[..., i].reshape(T // G, G) for i in range(3))
    dx, dy, dz = (db[..., i].reshape(T // G, G) for i in range(3))
    ln = lengths.reshape(T // G, G)

    idx_all, w_all = _point_math(ox, oy, oz, dx, dy, dz, ln)
    idx1 = idx_all.reshape(NCHUNK * 2 * C)
    w1 = w_all.reshape(NCHUNK * 8 * C)

    d, r, gg, b = _sc_gather()(table, idx1, w1)

    d_s = d.reshape(B, N, P, 1)
    f_s = jnp.stack([r, gg, b], axis=-1).reshape(B, N, P, 3)
    return (d_s, f_s)


# speedup vs baseline: 1.6394x; 1.0021x over previous
import functools

import jax
import jax.numpy as jnp
from jax import lax
from jax.experimental import pallas as pl
from jax.experimental.pallas import tpu as pltpu
from jax.experimental.pallas import tpu_sc as plsc

G = 128
V = G * G * G
B, N, P = 4, 4096, 64
T = B * N * P
C = 1024
NCHUNK = T // C
NCORES = 2
NSUB = 16
NW = NCORES * NSUB
CPW = NCHUNK // NW
LANES = 16


def _softplus_body(x_ref, o_ref):
    o_ref[...] = jnp.logaddexp(x_ref[...], 0.0)


def _sigmoid_body(x_ref, o_ref):
    o_ref[...] = jax.nn.sigmoid(x_ref[...])


def _activate(x2d, body):
    rows, cols = x2d.shape
    blk = 512
    return pl.pallas_call(
        body,
        grid=(rows // blk,),
        in_specs=[pl.BlockSpec((blk, cols), lambda g: (g, 0))],
        out_specs=pl.BlockSpec((blk, cols), lambda g: (g, 0)),
        out_shape=jax.ShapeDtypeStruct((rows, cols), jnp.float32),
    )(x2d)


_ROWS_PER_STEP = 64


def _points_body(ox, oy, oz, dx, dy, dz, ln, idx_ref, w_ref):
    t = ln[...]
    x = ox[...] + dx[...] * t
    y = oy[...] + dy[...] * t
    z = oz[...] + dz[...] * t
    gf = jnp.float32(G)
    ix = ((x + 1.0) * gf - 1.0) * 0.5
    iy = ((y + 1.0) * gf - 1.0) * 0.5
    iz = ((z + 1.0) * gf - 1.0) * 0.5
    ix0 = jnp.floor(ix)
    iy0 = jnp.floor(iy)
    iz0 = jnp.floor(iz)
    fx = ix - ix0
    fy = iy - iy0
    fz = iz - iz0
    shape3 = (_ROWS_PER_STEP // 8, 8, G)
    x0 = jnp.clip(ix0, -1, G - 1).astype(jnp.int32)
    y0 = jnp.clip(iy0, -1, G - 1).astype(jnp.int32)
    for zh in (0, 1):
        zc = jnp.clip(iz0 + zh, 0, G - 1).astype(jnp.int32)
        j = (zc * G + y0) * G + x0 + BIAS
        idx_ref[:, zh] = j.reshape(shape3)
    for a in (0, 1):
        for b in (0, 1):
            for c in (0, 1):
                k = a * 4 + b * 2 + c
                zi = iz0 + a
                yi = iy0 + b
                xi = ix0 + c
                wz = fz if a else 1.0 - fz
                wy = fy if b else 1.0 - fy
                wx = fx if c else 1.0 - fx
                valid = ((xi >= 0) & (xi < gf) & (yi >= 0) & (yi < gf)
                         & (zi >= 0) & (zi < gf))
                w_ref[:, k] = jnp.where(valid, wx * wy * wz, 0.0).reshape(shape3)


def _point_math(ox, oy, oz, dx, dy, dz, ln):
    nrows = T // G
    nsteps = nrows // _ROWS_PER_STEP
    cps = _ROWS_PER_STEP // 8
    in_spec = pl.BlockSpec((_ROWS_PER_STEP, G), lambda g: (g, 0))
    return pl.pallas_call(
        _points_body,
        grid=(nsteps,),
        in_specs=[in_spec] * 7,
        out_specs=[
            pl.BlockSpec((cps, 2, 8, G), lambda g: (g, 0, 0, 0)),
            pl.BlockSpec((cps, 8, 8, G), lambda g: (g, 0, 0, 0)),
        ],
        out_shape=[
            jax.ShapeDtypeStruct((NCHUNK, 2, 8, G), jnp.int32),
            jax.ShapeDtypeStruct((NCHUNK, 8, 8, G), jnp.float32),
        ],
    )(ox, oy, oz, dx, dy, dz, ln)


RB = 2048
MP = NW * 33 * RB
RPW = MP // NW
CH = 2184
BIAS = 136


def _sc_table_body(p0_hbm, p1_hbm, p2_hbm, p3_hbm, tbl_hbm, stage_v, tbl_v,
                   sem):
    wid = lax.axis_index("s") * NCORES + lax.axis_index("c")
    lanes = lax.iota(jnp.int32, LANES)
    ch_l = lanes & 3
    off_l = (lanes >> 3) * G + ((lanes >> 2) & 1)
    cvec = ch_l * CH + off_l
    srcs = (p0_hbm, p1_hbm, p2_hbm, p3_hbm)
    r0w = wid * RPW

    @pl.loop(0, RPW + RB, step=2 * RB)
    def _blk2(rr0):
        for b in (0, 1):
            rr = rr0 + b * RB

            @pl.when(rr < RPW)
            def _():
                r0 = r0w + rr
                lo = r0 - BIAS

                @pl.when(rr0 > 0)
                def _():
                    pltpu.make_async_copy(
                        tbl_v.at[b], tbl_hbm.at[pl.ds(0, RB * LANES)],
                        sem).wait()

                @pl.when(r0 == 0)
                def _():
                    @plsc.parallel_loop(0, BIAS, step=LANES)
                    def _z(i):
                        for c4 in range(4):
                            stage_v[pl.ds(c4 * CH + i, LANES)] = (
                                jnp.zeros((LANES,), jnp.float32))
                    for c4 in range(4):
                        pltpu.sync_copy(
                            srcs[c4].at[pl.ds(0, CH - BIAS)],
                            stage_v.at[pl.ds(c4 * CH + BIAS, CH - BIAS)])

                @pl.when((r0 > 0) & (r0 + RB <= V))
                def _():
                    for c4 in range(4):
                        pltpu.sync_copy(srcs[c4].at[pl.ds(lo, CH)],
                                        stage_v.at[pl.ds(c4 * CH, CH)])

                @pl.when(r0 == V)
                def _():
                    @plsc.parallel_loop(0, CH, step=LANES)
                    def _z(i):
                        for c4 in range(4):
                            stage_v[pl.ds(c4 * CH + i, LANES)] = (
                                jnp.zeros((LANES,), jnp.float32))
                    for c4 in range(4):
                        pltpu.sync_copy(srcs[c4].at[pl.ds(V - BIAS, BIAS)],
                                        stage_v.at[pl.ds(c4 * CH, BIAS)])

                @plsc.parallel_loop(0, RB, step=1, unroll=8)
                def _row(j2):
                    row = plsc.load_gather(stage_v, [cvec + j2])
                    tbl_v[b, pl.ds(j2 * LANES, LANES)] = row

                pltpu.async_copy(tbl_v.at[b],
                                 tbl_hbm.at[pl.ds(r0 * LANES, RB * LANES)],
                                 sem)

    for _ in range(2):
        pltpu.make_async_copy(
            tbl_v.at[0], tbl_hbm.at[pl.ds(0, RB * LANES)], sem).wait()


@functools.lru_cache(maxsize=None)
def _sc_table():
    return pl.kernel(
        _sc_table_body,
        out_type=jax.ShapeDtypeStruct((MP * LANES,), jnp.float32),
        mesh=plsc.VectorSubcoreMesh(
            core_axis_name="c", subcore_axis_name="s",
            num_cores=NCORES, num_subcores=NSUB),
        compiler_params=pltpu.CompilerParams(
            needs_layout_passes=False, use_tc_tiling_on_sc=False),
        scratch_types=[
            pltpu.VMEM((4 * CH + LANES,), jnp.float32),
            pltpu.VMEM((2, RB * LANES), jnp.float32),
            pltpu.SemaphoreType.DMA,
        ],
    )


def _lane_perm(x, idx):
    return lax.gather(
        x, idx[:, None],
        lax.GatherDimensionNumbers(
            offset_dims=(), collapsed_slice_dims=(0,), start_index_map=(0,)),
        slice_sizes=(1,),
        mode=lax.GatherScatterMode.PROMISE_IN_BOUNDS)


def _sc_body(table_hbm, idx_hbm, w_hbm,
             out_d, out_r, out_g, out_b, idx_v, ws_v, w_v, g_v, o4_v, sem):
    wid = lax.axis_index("s") * NCORES + lax.axis_index("c")
    lanes = lax.iota(jnp.int32, LANES)
    perm4 = (lanes + 4) & 15
    perm8 = (lanes + 8) & 15
    permw = lanes >> 2
    pats = [((lanes & 3) << 2) + ch for ch in range(4)]
    l9 = lanes * 9
    m4 = lanes < 4
    m8 = lanes < 8
    m12 = lanes < 12
    outs = (out_d, out_r, out_g, out_b)

    def _fetch(jc, b):
        g = wid * CPW + jc
        pltpu.sync_copy(idx_hbm.at[pl.ds(g * 2 * C, 2 * C)], idx_v.at[b])
        pltpu.async_copy(table_hbm.at[idx_v.at[b]], g_v.at[b], sem)

    _fetch(0, 0)

    @pl.loop(0, CPW, step=2)
    def _chunk2(jc0):
        for b in (0, 1):
            jc = jc0 + b
            g = wid * CPW + jc
            g_b = g_v.at[b]
            pltpu.make_async_copy(table_hbm.at[idx_v.at[b]], g_b, sem).wait()

            @pl.when(jc + 1 < CPW)
            def _():
                _fetch(jc + 1, 1 - b)

            pltpu.sync_copy(w_hbm.at[pl.ds(g * 8 * C, 8 * C)], ws_v)
            _compute_chunk(g, g_b, ws_v, w_v, o4_v, outs,
                           perm4, perm8, permw, pats, l9, m4, m8, m12)


def _compute_chunk(g, g_b, ws_v, w_v, o4_v, outs,
                   perm4, perm8, permw, pats, l9, m4, m8, m12):
    if True:
        @plsc.parallel_loop(0, C, step=LANES)
        def _tr(p0):
            for k in range(8):
                v = ws_v[pl.ds(k * C + p0, LANES)]
                plsc.store_scatter(w_v, [l9 + (9 * p0 + k)], v)

        @plsc.parallel_loop(0, C, step=16, unroll=2)
        def _pts(p):
            combs = []
            for q4 in range(4):
                t2 = []
                for q in range(4):
                    pq = p + q4 * 4 + q
                    a = g_b[pq, :]
                    b = g_b[C + pq, :]
                    wv = w_v[pl.ds(9 * pq, LANES)]
                    wa = _lane_perm(wv, permw)
                    wb = _lane_perm(wv, permw + 4)
                    s = a * wa + b * wb
                    t1 = s + _lane_perm(s, perm8)
                    t2.append(t1 + _lane_perm(t1, perm4))
                combs.append(jnp.where(m4, t2[0],
                                       jnp.where(m8, t2[1],
                                                 jnp.where(m12, t2[2], t2[3]))))
            for ch in range(4):
                e = [_lane_perm(cb, pats[ch]) for cb in combs]
                v16 = jnp.where(m4, e[0],
                                jnp.where(m8, e[1],
                                          jnp.where(m12, e[2], e[3])))
                o4_v[ch, pl.ds(p, LANES)] = v16

        for ch in range(4):
            pltpu.sync_copy(o4_v.at[ch], outs[ch].at[pl.ds(g * C, C)])


@functools.lru_cache(maxsize=None)
def _sc_gather():
    return pl.kernel(
        _sc_body,
        out_type=[jax.ShapeDtypeStruct((T,), jnp.float32)] * 4,
        mesh=plsc.VectorSubcoreMesh(
            core_axis_name="c", subcore_axis_name="s",
            num_cores=NCORES, num_subcores=NSUB),
        compiler_params=pltpu.CompilerParams(
            needs_layout_passes=False, use_tc_tiling_on_sc=False),
        scratch_types=[
            pltpu.VMEM((2, 2 * C), jnp.int32),
            pltpu.VMEM((8 * C,), jnp.float32),
            pltpu.VMEM((9 * C + LANES,), jnp.float32),
            pltpu.VMEM((2, 2 * C, LANES), jnp.float32),
            pltpu.VMEM((4, C), jnp.float32),
            pltpu.SemaphoreType.DMA,
        ],
    )


def kernel(density, color, origins, directions, lengths):
    d_act = _activate(density.reshape(V // G, G), _softplus_body)
    c_act = _activate(color.reshape(3 * V // G, G),
                      _sigmoid_body).reshape(3, V // G, G)

    chans = (d_act.reshape(V), c_act[0].reshape(V),
             c_act[1].reshape(V), c_act[2].reshape(V))
    table = _sc_table()(*chans).reshape(MP, LANES)

    ob = jnp.broadcast_to(origins[:, :, None, :], (B, N, P, 3))
    db = jnp.broadcast_to(directions[:, :, None, :], (B, N, P, 3))
    ox, oy, oz = (ob[..., i].reshape(T // G, G) for i in range(3))
    dx, dy, dz = (db[..., i].reshape(T // G, G) for i in range(3))
    ln = lengths.reshape(T // G, G)

    idx_all, w_all = _point_math(ox, oy, oz, dx, dy, dz, ln)
    idx1 = idx_all.reshape(NCHUNK * 2 * C)
    w1 = w_all.reshape(NCHUNK * 8 * C)

    d, r, gg, b = _sc_gather()(table, idx1, w1)

    d_s = d.reshape(B, N, P, 1)
    f_s = jnp.stack([r, gg, b], axis=-1).reshape(B, N, P, 3)
    return (d_s, f_s)
